# baseline (speedup 1.0000x reference)
import jax
import jax.numpy as jnp
from jax import lax
from jax.experimental import pallas as pl
from jax.experimental.pallas import tpu as pltpu
from jax.experimental.pallas import tpu_sc as plsc

L = 16
NC = 2
NS = 16
NW = NC * NS

B, R, C = 8, 20000, 128
G = C // L
WPB = NW // B
SPAN = 5120
CHUNK = 256
NCHUNK = SPAN // CHUNK
GPC = CHUNK // L

BIG = 0x7F000000


def _worker_span(q):
  start = q * 5000 + 8 * (q % 2)
  start = jnp.where(q == WPB - 1, R - SPAN, start)
  return pl.multiple_of(start, 16)


def _iota():
  return lax.broadcasted_iota(jnp.int32, (L,), 0)


_GDN = lax.GatherDimensionNumbers(
    offset_dims=(), collapsed_slice_dims=(0,), start_index_map=(0,))


def _perm(x, idx):
  return lax.gather(x, idx[:, None], _GDN, (1,),
                    mode=lax.GatherScatterMode.PROMISE_IN_BOUNDS)


def _kernel_a(sim, rowmax, rowarg, pval, prow, buf, ovbuf, oibuf, pvbuf, prbuf):
  wid = lax.axis_index("s") * NC + lax.axis_index("c")
  b = wid // WPB
  q = wid % WPB
  start = _worker_span(q)

  iota = _iota()
  idx_c = [iota + L * g for g in range(G)]
  onehot = [iota == r for r in range(L)]
  bigv = jnp.full((L,), BIG, jnp.int32)
  perms = [iota ^ s for s in (1, 2, 4, 8)]
  padv = jnp.full((L,), -1.0, jnp.float32)
  padi = jnp.full((L,), C, jnp.int32)

  def chunk_body(ch, carry):
    cmax, crow = carry
    row0 = pl.multiple_of(start + ch * CHUNK, 16)
    pltpu.sync_copy(sim.at[b, pl.ds(row0, CHUNK)], buf)

    def grp_body(gi, carry2):
      cmax = list(carry2[0])
      crow = list(carry2[1])
      outv = jnp.zeros((L,), jnp.float32)
      outi = jnp.zeros((L,), jnp.int32)
      for r in range(L):
        rloc = gi * L + r
        v = [buf[rloc, pl.ds(L * g, L)] for g in range(G)]
        rowvec = jnp.full((L,), row0 + rloc, jnp.int32)
        for g in range(G):
          m = v[g] > cmax[g]
          cmax[g] = jnp.where(m, v[g], cmax[g])
          crow[g] = jnp.where(m, rowvec, crow[g])
        val = list(v)
        idx = list(idx_c)
        n = G
        while n > 1:
          nv, ni = [], []
          for p in range(0, n, 2):
            m = val[p + 1] > val[p]
            nv.append(jnp.where(m, val[p + 1], val[p]))
            ni.append(jnp.where(m, idx[p + 1], idx[p]))
          val, idx, n = nv, ni, n // 2
        mx = val[0]
        for p in perms:
          mx = jnp.maximum(mx, _perm(mx, p))
        am = jnp.where(val[0] == mx, idx[0], bigv)
        for p in perms:
          am = jnp.minimum(am, _perm(am, p))
        padw = mx < padv
        mx = jnp.where(padw, padv, mx)
        am = jnp.where(padw, padi, am)
        outv = jnp.where(onehot[r], mx, outv)
        outi = jnp.where(onehot[r], am, outi)
      off = ch * CHUNK + gi * L
      ovbuf[pl.ds(off, L)] = outv
      oibuf[pl.ds(off, L)] = outi
      return tuple(cmax), tuple(crow)

    return lax.fori_loop(0, GPC, grp_body, (cmax, crow))

  cmax0 = tuple(jnp.full((L,), -jnp.inf, jnp.float32) for _ in range(G))
  crow0 = tuple(jnp.zeros((L,), jnp.int32) for _ in range(G))
  cmax, crow = lax.fori_loop(0, NCHUNK, chunk_body, (cmax0, crow0))

  for g in range(G):
    pvbuf[pl.ds(L * g, L)] = cmax[g]
    prbuf[pl.ds(L * g, L)] = crow[g]
  rofs = pl.multiple_of(b * R + start, 16)
  pofs = pl.multiple_of(wid * C, 16)
  pltpu.sync_copy(ovbuf, rowmax.at[pl.ds(rofs, SPAN)])
  pltpu.sync_copy(oibuf, rowarg.at[pl.ds(rofs, SPAN)])
  pltpu.sync_copy(pvbuf, pval.at[pl.ds(pofs, C)])
  pltpu.sync_copy(prbuf, prow.at[pl.ds(pofs, C)])


def _kernel_b(rowmax, rowarg, pval, prow, mcols, mvals,
              table, rmbuf, rabuf, mcbuf, mvbuf, pvb, prb):
  wid = lax.axis_index("s") * NC + lax.axis_index("c")
  b = wid // WPB
  q = wid % WPB
  start = _worker_span(q)

  iota = _iota()
  idx_c = [iota + L * g for g in range(G)]
  onehot = [iota == r for r in range(L)]
  bigv = jnp.full((L,), BIG, jnp.int32)

  pofs = pl.multiple_of(b * WPB * C, 16)
  pltpu.sync_copy(pval.at[pl.ds(pofs, WPB * C)], pvb)
  pltpu.sync_copy(prow.at[pl.ds(pofs, WPB * C)], prb)
  mrow = []
  for g in range(G):
    cur = pvb[pl.ds(L * g, L)]
    curr = prb[pl.ds(L * g, L)]
    for k in range(1, WPB):
      vk = pvb[pl.ds(k * C + L * g, L)]
      rk = prb[pl.ds(k * C + L * g, L)]
      m = vk > cur
      cur = jnp.where(m, vk, cur)
      curr = jnp.where(m, rk, curr)
    mrow.append(curr)

  def init_body(i, _):
    table[pl.ds(i * L, L)] = bigv
    return 0
  lax.fori_loop(0, SPAN // L, init_body, 0)

  startv = jnp.full((L,), start, jnp.int32)
  pad_idx = jnp.zeros((L,), jnp.int32) - startv
  pad_in = (pad_idx >= 0) & (pad_idx < SPAN)
  plsc.store_scatter(table, [pad_idx], jnp.full((L,), C, jnp.int32),
                     mask=pad_in & onehot[0])
  for g in reversed(range(G)):
    rcl = mrow[g] - startv
    inr = (rcl >= 0) & (rcl < SPAN)
    for lane in reversed(range(L)):
      plsc.store_scatter(table, [rcl], idx_c[g], mask=inr & onehot[lane])

  rofs = pl.multiple_of(b * R + start, 16)
  pltpu.sync_copy(rowmax.at[pl.ds(rofs, SPAN)], rmbuf)
  pltpu.sync_copy(rowarg.at[pl.ds(rofs, SPAN)], rabuf)
  one = jnp.full((L,), 1, jnp.int32)
  neg1 = jnp.full((L,), -1, jnp.int32)
  neg2 = jnp.full((L,), -2, jnp.int32)

  def comb_body(i, _):
    sl = pl.ds(i * L, L)
    f = table[sl]
    forced = f < bigv
    rm = rmbuf[sl]
    cls = jnp.where(rm >= jnp.float32(0.5), one,
                    jnp.where(rm >= jnp.float32(0.4), neg2, neg1))
    mcbuf[sl] = jnp.where(forced, f, rabuf[sl])
    mvbuf[sl] = jnp.where(forced, one, cls)
    return 0
  lax.fori_loop(0, SPAN // L, comb_body, 0)

  pltpu.sync_copy(mcbuf, mcols.at[pl.ds(rofs, SPAN)])
  pltpu.sync_copy(mvbuf, mvals.at[pl.ds(rofs, SPAN)])


def kernel(similarity_matrix):
  assert similarity_matrix.shape == (B, R, C)
  mesh = plsc.VectorSubcoreMesh(core_axis_name="c", subcore_axis_name="s")
  params = pltpu.CompilerParams(needs_layout_passes=False)

  rowmax, rowarg, pval, prow = pl.kernel(
      _kernel_a,
      out_type=[
          jax.ShapeDtypeStruct((B * R,), jnp.float32),
          jax.ShapeDtypeStruct((B * R,), jnp.int32),
          jax.ShapeDtypeStruct((NW * C,), jnp.float32),
          jax.ShapeDtypeStruct((NW * C,), jnp.int32),
      ],
      mesh=mesh,
      scratch_types=[
          pltpu.VMEM((CHUNK, C), jnp.float32),
          pltpu.VMEM((SPAN,), jnp.float32),
          pltpu.VMEM((SPAN,), jnp.int32),
          pltpu.VMEM((C,), jnp.float32),
          pltpu.VMEM((C,), jnp.int32),
      ],
      compiler_params=params,
  )(similarity_matrix)

  mcols, mvals = pl.kernel(
      _kernel_b,
      out_type=[
          jax.ShapeDtypeStruct((B * R,), jnp.int32),
          jax.ShapeDtypeStruct((B * R,), jnp.int32),
      ],
      mesh=mesh,
      scratch_types=[
          pltpu.VMEM((SPAN,), jnp.int32),
          pltpu.VMEM((SPAN,), jnp.float32),
          pltpu.VMEM((SPAN,), jnp.int32),
          pltpu.VMEM((SPAN,), jnp.int32),
          pltpu.VMEM((SPAN,), jnp.int32),
          pltpu.VMEM((WPB * C,), jnp.float32),
          pltpu.VMEM((WPB * C,), jnp.int32),
      ],
      compiler_params=params,
  )(rowmax, rowarg, pval, prow)

  return mcols.reshape(B, R), mvals.reshape(B, R)
